# baseline (speedup 1.0000x reference)
import functools

import jax
import jax.numpy as jnp
from jax import lax
from jax.experimental import pallas as pl
from jax.experimental.pallas import tpu as pltpu
from jax.experimental.pallas import tpu_sc as plsc

N_NODES = 10000
N_HE = 20000
K = 8
D_IN = 128
D_HID = 64
N_CLS = 16

NC, NS = 2, 16
NW = NC * NS
HE_PAD = 20480
HE_W = HE_PAD // NW
HE_S = HE_PAD // NS
NG = HE_W // 16
NG1 = HE_S // 16
N_PAD = 10240
ROWS_W = N_PAD // NS
INV_C = 1.0 / (2.0 * K - 3.0)

_MESH = plsc.VectorSubcoreMesh(core_axis_name="c", subcore_axis_name="s")


_NB = 4
_BR = N_PAD // _NB


def _mmq_body(h_ref, w_ref, rv_ref, hw_ref, q_ref):
    hw = jnp.dot(h_ref[...], w_ref[...], preferred_element_type=jnp.float32)
    hw_ref[...] = hw
    q_ref[...] = jnp.dot(hw, rv_ref[...], preferred_element_type=jnp.float32)


def _mmq(h, w, rv):
    n, (k, d) = h.shape[0], w.shape
    return pl.pallas_call(
        _mmq_body,
        grid=(_NB,),
        in_specs=[pl.BlockSpec((_BR, k), lambda i: (i, 0)),
                  pl.BlockSpec((k, d), lambda i: (0, 0)),
                  pl.BlockSpec((d, 1), lambda i: (0, 0))],
        out_specs=[pl.BlockSpec((_BR, d), lambda i: (i, 0)),
                   pl.BlockSpec((_BR, 1), lambda i: (i, 0))],
        out_shape=[jax.ShapeDtypeStruct((n, d), jnp.float32),
                   jax.ShapeDtypeStruct((n, 1), jnp.float32)],
    )(h, w, rv.reshape(-1, 1))


def _mid_body(ap_ref, hw_ref, dv_ref, b_ref, w_ref, rv_ref, hw2_ref, q2_ref):
    dv = dv_ref[...]
    a = ap_ref[0] + ap_ref[1] + dv * dv * hw_ref[...] + b_ref[...]
    h1 = jnp.maximum(a, 0.0)
    hw2 = jnp.dot(h1, w_ref[...], preferred_element_type=jnp.float32)
    hw2_ref[...] = hw2
    q2_ref[...] = jnp.dot(hw2, rv_ref[...], preferred_element_type=jnp.float32)


def _mid(ap, hw, dinv, b, w, rv):
    n = ap.shape[1]
    d, d2 = w.shape
    return pl.pallas_call(
        _mid_body,
        grid=(_NB,),
        in_specs=[pl.BlockSpec((NC, _BR, d), lambda i: (0, i, 0)),
                  pl.BlockSpec((_BR, d), lambda i: (i, 0)),
                  pl.BlockSpec((_BR, 1), lambda i: (i, 0)),
                  pl.BlockSpec((1, d), lambda i: (0, 0)),
                  pl.BlockSpec((d, d2), lambda i: (0, 0)),
                  pl.BlockSpec((d2, 1), lambda i: (0, 0))],
        out_specs=[pl.BlockSpec((_BR, d2), lambda i: (i, 0)),
                   pl.BlockSpec((_BR, 1), lambda i: (i, 0))],
        out_shape=[jax.ShapeDtypeStruct((n, d2), jnp.float32),
                   jax.ShapeDtypeStruct((n, 1), jnp.float32)],
    )(ap, hw, dinv.reshape(-1, 1), b.reshape(1, -1), w, rv.reshape(-1, 1))


def _fin_body(ap_ref, hw_ref, dv_ref, b_ref, out_ref):
    dv = dv_ref[...]
    a = ap_ref[0] + ap_ref[1] + dv * dv * hw_ref[...] + b_ref[...]
    h2 = jnp.maximum(a, 0.0)
    z = h2 - jnp.max(h2, axis=1, keepdims=True)
    out_ref[...] = z - jnp.log(jnp.sum(jnp.exp(z), axis=1, keepdims=True))


def _fin(ap, hw, dinv, b):
    n, d = ap.shape[1], ap.shape[2]
    return pl.pallas_call(
        _fin_body,
        grid=(_NB,),
        in_specs=[pl.BlockSpec((NC, _BR, d), lambda i: (0, i, 0)),
                  pl.BlockSpec((_BR, d), lambda i: (i, 0)),
                  pl.BlockSpec((_BR, 1), lambda i: (i, 0)),
                  pl.BlockSpec((1, d), lambda i: (0, 0))],
        out_specs=pl.BlockSpec((_BR, d), lambda i: (i, 0)),
        out_shape=jax.ShapeDtypeStruct((n, d), jnp.float32),
    )(ap, hw, dinv.reshape(-1, 1), b.reshape(1, -1))


def _rsqrt16(x):
    i = plsc.bitcast(x, jnp.int32)
    i = 0x5F3759DF - lax.shift_right_logical(i, 1)
    y = plsc.bitcast(i, jnp.float32)
    for _ in range(3):
        y = y * (1.5 - 0.5 * x * y * y)
    return y


def _slayer_body(d, ew_ref, q_ref, hw_ref, ap_ref, dinv_ref,
                 qbuf, etbuf, etbuf3, sxbuf, ixbuf,
                 idxm, valm, idxp, valp, idxm1, valm1, idxp1, valp1,
                 idxm2, valm2, idxp2, valp2, idxm3, valm3, idxp3, valp3,
                 dvbuf, dinvfull,
                 idx0, idx1, sidx0, sidx1, rm0, rm1, sm0, sm1,
                 abuf, ambuf, apbuf, sem0, sem1, ssem0, ssem1,
                 dsem0, dsem1, dsem2, dsem3, deg_sh, sx_sh, ix_sh, acc_sh):
    nch = d // 16
    unroll = 4 if nch == 1 else 2
    c = lax.axis_index("c")
    s = lax.axis_index("s")
    wid = c * NS + s
    iota = lax.iota(jnp.int32, 16)
    z16 = jnp.zeros((16,), jnp.int32)

    def zrow(r, carry):
        for ch in range(nch):
            sm0[r, pl.ds(ch * 16, 16)] = jnp.zeros((16,), jnp.float32)
        return carry

    lax.fori_loop(0, 128, zrow, 0)
    for i in range(ROWS_W // 128):
        pltpu.sync_copy(sm0, acc_sh.at[pl.ds(s * ROWS_W + i * 128, 128), :])
    for i in range(ROWS_W // 16):
        dvbuf[pl.ds(i * 16, 16)] = jnp.zeros((16,), jnp.float32)
    pltpu.sync_copy(dvbuf, deg_sh.at[pl.ds(s * ROWS_W, ROWS_W)])
    plsc.subcore_barrier()

    pltpu.sync_copy(q_ref, qbuf)
    pltpu.sync_copy(ew_ref.at[2 * s], etbuf.at[:, pl.ds(0, HE_W)])
    pltpu.sync_copy(ew_ref.at[2 * s + 1], etbuf.at[:, pl.ds(HE_W, HE_W)])

    dsets = ((idxm, valm, idxp, valp, dsem0),
             (idxm1, valm1, idxp1, valp1, dsem1),
             (idxm2, valm2, idxp2, valp2, dsem2),
             (idxm3, valm3, idxp3, valp3, dsem3))

    def group1(gg, carry):
        for p in range(4):
            g = 4 * gg + p
            base = g * 16
            im, vm, ip, vp_b, dsem = dsets[p]

            @pl.when(gg > 0)
            def _():
                pltpu.make_async_copy(vm, deg_sh.at[im], dsem).wait()
                pltpu.make_async_copy(vp_b, deg_sh.at[ip], dsem).wait()

            idxs = [etbuf[j, pl.ds(base, 16)] for j in range(K)]
            ps = [plsc.load_gather(qbuf, [idxs[j]]) for j in range(K)]
            mx, se = ps[0], idxs[0]
            mn, ie = ps[0], idxs[0]
            sarg = jnp.zeros((16,), jnp.int32)
            iarg = jnp.zeros((16,), jnp.int32)
            for j in range(1, K):
                up = ps[j] > mx
                mx = jnp.where(up, ps[j], mx)
                se = jnp.where(up, idxs[j], se)
                sarg = jnp.where(up, j, sarg)
                dn = ps[j] < mn
                mn = jnp.where(dn, ps[j], mn)
                ie = jnp.where(dn, idxs[j], ie)
                iarg = jnp.where(dn, j, iarg)
            sxbuf[pl.ds(base, 16)] = sarg
            ixbuf[pl.ds(base, 16)] = iarg
            nm = jnp.zeros((16,), jnp.float32)
            for j in range(K):
                m = jnp.where((idxs[j] != se) & (idxs[j] != ie), 1.0, 0.0)
                nm = nm + m
                im[pl.ds(j * 16, 16)] = idxs[j]
                vm[pl.ds(j * 16, 16)] = m * (2.0 * INV_C)
            vp = (1.0 + nm) * INV_C
            ip[pl.ds(0, 16)] = se
            vp_b[pl.ds(0, 16)] = vp
            ip[pl.ds(16, 16)] = ie
            vp_b[pl.ds(16, 16)] = vp
            pltpu.async_copy(vm, deg_sh.at[im], dsem, add=True)
            pltpu.async_copy(vp_b, deg_sh.at[ip], dsem, add=True)
        return carry

    lax.fori_loop(0, NG1 // 4, group1, 0)
    for im, vm, ip, vp_b, dsem in dsets:
        pltpu.make_async_copy(vm, deg_sh.at[im], dsem).wait()
        pltpu.make_async_copy(vp_b, deg_sh.at[ip], dsem).wait()
    pltpu.sync_copy(sxbuf.at[pl.ds(0, HE_S)], sx_sh.at[pl.ds(s * HE_S, HE_S)])
    pltpu.sync_copy(ixbuf.at[pl.ds(0, HE_S)], ix_sh.at[pl.ds(s * HE_S, HE_S)])
    plsc.subcore_barrier()

    pltpu.sync_copy(deg_sh.at[pl.ds(s * ROWS_W, ROWS_W)], dvbuf)

    def dloop(i, carry):
        dvbuf[pl.ds(i * 16, 16)] = _rsqrt16(1.0 + dvbuf[pl.ds(i * 16, 16)])
        return carry

    lax.fori_loop(0, ROWS_W // 16, dloop, 0)
    pltpu.sync_copy(dvbuf, deg_sh.at[pl.ds(s * ROWS_W, ROWS_W)])

    @pl.when(c == 0)
    def _():
        pltpu.sync_copy(dvbuf, dinv_ref.at[pl.ds(s * ROWS_W, ROWS_W)])

    plsc.subcore_barrier()
    pltpu.sync_copy(deg_sh, dinvfull)

    pltpu.sync_copy(ew_ref.at[wid], etbuf3)
    pltpu.sync_copy(sx_sh.at[pl.ds(wid * HE_W, HE_W)], sxbuf.at[pl.ds(0, HE_W)])
    pltpu.sync_copy(ix_sh.at[pl.ds(wid * HE_W, HE_W)], ixbuf.at[pl.ds(0, HE_W)])

    def start_gather(g, idx, rm, sem):
        base = g * 16
        for j in range(K):
            idx[pl.ds(j * 16, 16)] = etbuf3[j, pl.ds(base, 16)]
        pltpu.async_copy(hw_ref.at[idx], rm, sem)

    def compute_group(t, g, idx, rm, sm, sidx, ssem):
        base = g * 16
        sx = sxbuf[pl.ds(base, 16)]
        ix = ixbuf[pl.ds(base, 16)]
        se = plsc.load_gather(etbuf3, [sx, base + iota])
        ie = plsc.load_gather(etbuf3, [ix, base + iota])
        for j in range(K):
            vj = etbuf3[j, pl.ds(base, 16)]
            dj = plsc.load_gather(dinvfull, [vj])
            m = jnp.where((vj != se) & (vj != ie), 1.0, 0.0)
            ambuf[j, :] = m * dj
        apbuf[0, :] = plsc.load_gather(dinvfull, [se])
        apbuf[1, :] = plsc.load_gather(dinvfull, [ie])

        @pl.when(t > 0)
        def _():
            pltpu.make_async_copy(sm, acc_sh.at[sidx], ssem).wait()

        def he(hu, inner):
            for u in range(unroll):
                h = hu * unroll + u
                s_h = sxbuf[pl.ds(base + h, 16)][0]
                i_h = ixbuf[pl.ds(base + h, 16)][0]
                rs = s_h * 16 + h
                ri = i_h * 16 + h
                h_vec = z16 + h
                dse_b = plsc.load_gather(apbuf, [z16, h_vec])
                die_b = plsc.load_gather(apbuf, [z16 + 1, h_vec])
                am_j = [plsc.load_gather(ambuf, [z16 + j, h_vec])
                        for j in range(K)]
                for ch in range(nch):
                    sl = pl.ds(ch * 16, 16)
                    gse = dse_b * rm[rs, sl]
                    gie = die_b * rm[ri, sl]
                    pairc = (gse + gie) * INV_C
                    msum = jnp.zeros((16,), jnp.float32)
                    for j in range(K):
                        msum = msum + am_j[j] * rm[j * 16 + h, sl]
                    for j in range(K):
                        sm[j * 16 + h, sl] = am_j[j] * pairc
                    sm[rs, sl] = dse_b * ((gie + msum) * INV_C)
                    prev = sm[ri, sl]
                    sm[ri, sl] = prev + die_b * ((gse + msum) * INV_C)
            return inner

        lax.fori_loop(0, 16 // unroll, he, 0)
        for j in range(K):
            sidx[pl.ds(j * 16, 16)] = idx[pl.ds(j * 16, 16)]
        pltpu.async_copy(sm, acc_sh.at[sidx], ssem, add=True)

    start_gather(0, idx0, rm0, sem0)

    def tbody(t, carry):
        g0 = 2 * t
        start_gather(g0 + 1, idx1, rm1, sem1)
        pltpu.make_async_copy(hw_ref.at[idx0], rm0, sem0).wait()
        compute_group(t, g0, idx0, rm0, sm0, sidx0, ssem0)

        @pl.when(t < NG // 2 - 1)
        def _():
            start_gather(g0 + 2, idx0, rm0, sem0)

        pltpu.make_async_copy(hw_ref.at[idx1], rm1, sem1).wait()
        compute_group(t, g0 + 1, idx1, rm1, sm1, sidx1, ssem1)
        return carry

    lax.fori_loop(0, NG // 2, tbody, 0)
    pltpu.make_async_copy(sm0, acc_sh.at[sidx0], ssem0).wait()
    pltpu.make_async_copy(sm1, acc_sh.at[sidx1], ssem1).wait()
    plsc.subcore_barrier()
    pltpu.sync_copy(acc_sh.at[pl.ds(s * ROWS_W, ROWS_W), :],
                    ap_ref.at[c].at[pl.ds(s * ROWS_W, ROWS_W), :])


def _slayer(ew, q, hw, d):
    f = pl.kernel(
        functools.partial(_slayer_body, d),
        out_type=[jax.ShapeDtypeStruct((NC, N_PAD, d), jnp.float32),
                  jax.ShapeDtypeStruct((N_PAD,), jnp.float32)],
        mesh=_MESH,
        compiler_params=pltpu.CompilerParams(needs_layout_passes=False,
                                             use_tc_tiling_on_sc=False),
        scratch_types=[
            pltpu.VMEM((N_PAD,), jnp.float32),
            pltpu.VMEM((K, HE_S), jnp.int32),
            pltpu.VMEM((K, HE_W), jnp.int32),
            pltpu.VMEM((HE_S + 16,), jnp.int32),
            pltpu.VMEM((HE_S + 16,), jnp.int32),
            pltpu.VMEM((K * 16,), jnp.int32),
            pltpu.VMEM((K * 16,), jnp.float32),
            pltpu.VMEM((32,), jnp.int32),
            pltpu.VMEM((32,), jnp.float32),
            pltpu.VMEM((K * 16,), jnp.int32),
            pltpu.VMEM((K * 16,), jnp.float32),
            pltpu.VMEM((32,), jnp.int32),
            pltpu.VMEM((32,), jnp.float32),
            pltpu.VMEM((K * 16,), jnp.int32),
            pltpu.VMEM((K * 16,), jnp.float32),
            pltpu.VMEM((32,), jnp.int32),
            pltpu.VMEM((32,), jnp.float32),
            pltpu.VMEM((K * 16,), jnp.int32),
            pltpu.VMEM((K * 16,), jnp.float32),
            pltpu.VMEM((32,), jnp.int32),
            pltpu.VMEM((32,), jnp.float32),
            pltpu.VMEM((ROWS_W,), jnp.float32),
            pltpu.VMEM((N_PAD,), jnp.float32),
            pltpu.VMEM((K * 16,), jnp.int32),
            pltpu.VMEM((K * 16,), jnp.int32),
            pltpu.VMEM((K * 16,), jnp.int32),
            pltpu.VMEM((K * 16,), jnp.int32),
            pltpu.VMEM((K * 16, d), jnp.float32),
            pltpu.VMEM((K * 16, d), jnp.float32),
            pltpu.VMEM((K * 16, d), jnp.float32),
            pltpu.VMEM((K * 16, d), jnp.float32),
            pltpu.VMEM((K, 16), jnp.float32),
            pltpu.VMEM((K, 16), jnp.float32),
            pltpu.VMEM((4, 16), jnp.float32),
            pltpu.SemaphoreType.DMA,
            pltpu.SemaphoreType.DMA,
            pltpu.SemaphoreType.DMA,
            pltpu.SemaphoreType.DMA,
            pltpu.SemaphoreType.DMA,
            pltpu.SemaphoreType.DMA,
            pltpu.SemaphoreType.DMA,
            pltpu.SemaphoreType.DMA,
            pltpu.VMEM_SHARED((N_PAD,), jnp.float32),
            pltpu.VMEM_SHARED((HE_PAD,), jnp.int32),
            pltpu.VMEM_SHARED((HE_PAD,), jnp.int32),
            pltpu.VMEM_SHARED((N_PAD, d), jnp.float32),
        ],
    )
    return f(ew, q, hw)


def kernel(E, H, W1, b1, W2, b2):
    key = jax.random.key(42)
    rv1 = jax.random.uniform(jax.random.fold_in(key, 0), (D_HID,),
                             dtype=jnp.float32)
    rv2 = jax.random.uniform(jax.random.fold_in(key, 1), (N_CLS,),
                             dtype=jnp.float32)
    h_pad = jnp.zeros((N_PAD, D_IN), jnp.float32).at[:N_NODES].set(H)
    dummy_cols = (jnp.arange(HE_PAD, dtype=jnp.int32) % 16) + N_NODES
    et = jnp.broadcast_to(dummy_cols, (K, HE_PAD))
    et = et.at[:, :N_HE].set(E.T.astype(jnp.int32))
    ew = et.reshape(K, NW, HE_W).transpose(1, 0, 2)

    hw1, q1 = _mmq(h_pad, W1, rv1)
    a1p, dinv1 = _slayer(ew, q1.reshape(N_PAD), hw1, D_HID)
    hw2, q2 = _mid(a1p, hw1, dinv1, b1, W2, rv2)
    a2p, dinv2 = _slayer(ew, q2.reshape(N_PAD), hw2, N_CLS)
    out = _fin(a2p, hw2, dinv2, b2)
    return out[:N_NODES]


# speedup vs baseline: 1.0115x; 1.0115x over previous
import functools

import jax
import jax.numpy as jnp
from jax import lax
from jax.experimental import pallas as pl
from jax.experimental.pallas import tpu as pltpu
from jax.experimental.pallas import tpu_sc as plsc

N_NODES = 10000
N_HE = 20000
K = 8
D_IN = 128
D_HID = 64
N_CLS = 16

NC, NS = 2, 16
NW = NC * NS
HE_PAD = 20480
HE_W = HE_PAD // NW
HE_S = HE_PAD // NS
NG = HE_W // 16
NG1 = HE_S // 16
N_PAD = 10240
ROWS_W = N_PAD // NS
INV_C = 1.0 / (2.0 * K - 3.0)

_MESH = plsc.VectorSubcoreMesh(core_axis_name="c", subcore_axis_name="s")


_NB = 4
_BR = N_PAD // _NB


def _mmq_body(h_ref, w_ref, rv_ref, hw_ref, q_ref):
    hw = jnp.dot(h_ref[...], w_ref[...], preferred_element_type=jnp.float32)
    hw_ref[...] = hw
    q_ref[...] = jnp.dot(hw, rv_ref[...], preferred_element_type=jnp.float32)


def _mmq(h, w, rv):
    n, (k, d) = h.shape[0], w.shape
    return pl.pallas_call(
        _mmq_body,
        grid=(_NB,),
        in_specs=[pl.BlockSpec((_BR, k), lambda i: (i, 0)),
                  pl.BlockSpec((k, d), lambda i: (0, 0)),
                  pl.BlockSpec((d, 1), lambda i: (0, 0))],
        out_specs=[pl.BlockSpec((_BR, d), lambda i: (i, 0)),
                   pl.BlockSpec((_BR, 1), lambda i: (i, 0))],
        out_shape=[jax.ShapeDtypeStruct((n, d), jnp.float32),
                   jax.ShapeDtypeStruct((n, 1), jnp.float32)],
    )(h, w, rv.reshape(-1, 1))


def _mid_body(ap_ref, hw_ref, dv_ref, b_ref, w_ref, rv_ref, hw2_ref, q2_ref):
    dv = dv_ref[...]
    a = ap_ref[0] + ap_ref[1] + dv * dv * hw_ref[...] + b_ref[...]
    h1 = jnp.maximum(a, 0.0)
    hw2 = jnp.dot(h1, w_ref[...], preferred_element_type=jnp.float32)
    hw2_ref[...] = hw2
    q2_ref[...] = jnp.dot(hw2, rv_ref[...], preferred_element_type=jnp.float32)


def _mid(ap, hw, dinv, b, w, rv):
    n = ap.shape[1]
    d, d2 = w.shape
    return pl.pallas_call(
        _mid_body,
        grid=(_NB,),
        in_specs=[pl.BlockSpec((NC, _BR, d), lambda i: (0, i, 0)),
                  pl.BlockSpec((_BR, d), lambda i: (i, 0)),
                  pl.BlockSpec((_BR, 1), lambda i: (i, 0)),
                  pl.BlockSpec((1, d), lambda i: (0, 0)),
                  pl.BlockSpec((d, d2), lambda i: (0, 0)),
                  pl.BlockSpec((d2, 1), lambda i: (0, 0))],
        out_specs=[pl.BlockSpec((_BR, d2), lambda i: (i, 0)),
                   pl.BlockSpec((_BR, 1), lambda i: (i, 0))],
        out_shape=[jax.ShapeDtypeStruct((n, d2), jnp.float32),
                   jax.ShapeDtypeStruct((n, 1), jnp.float32)],
    )(ap, hw, dinv.reshape(-1, 1), b.reshape(1, -1), w, rv.reshape(-1, 1))


def _fin_body(ap_ref, hw_ref, dv_ref, b_ref, out_ref):
    dv = dv_ref[...]
    a = ap_ref[0] + ap_ref[1] + dv * dv * hw_ref[...] + b_ref[...]
    h2 = jnp.maximum(a, 0.0)
    z = h2 - jnp.max(h2, axis=1, keepdims=True)
    out_ref[...] = z - jnp.log(jnp.sum(jnp.exp(z), axis=1, keepdims=True))


def _fin(ap, hw, dinv, b):
    n, d = ap.shape[1], ap.shape[2]
    return pl.pallas_call(
        _fin_body,
        grid=(_NB,),
        in_specs=[pl.BlockSpec((NC, _BR, d), lambda i: (0, i, 0)),
                  pl.BlockSpec((_BR, d), lambda i: (i, 0)),
                  pl.BlockSpec((_BR, 1), lambda i: (i, 0)),
                  pl.BlockSpec((1, d), lambda i: (0, 0))],
        out_specs=pl.BlockSpec((_BR, d), lambda i: (i, 0)),
        out_shape=jax.ShapeDtypeStruct((n, d), jnp.float32),
    )(ap, hw, dinv.reshape(-1, 1), b.reshape(1, -1))


def _rsqrt16(x):
    i = plsc.bitcast(x, jnp.int32)
    i = 0x5F3759DF - lax.shift_right_logical(i, 1)
    y = plsc.bitcast(i, jnp.float32)
    for _ in range(3):
        y = y * (1.5 - 0.5 * x * y * y)
    return y


def _slayer_body(d, ew_ref, q_ref, hw_ref, ap_ref, dinv_ref,
                 qbuf, etbuf, etbuf3, sxbuf, ixbuf,
                 idxm, valm, idxp, valp, idxm1, valm1, idxp1, valp1,
                 idxm2, valm2, idxp2, valp2, idxm3, valm3, idxp3, valp3,
                 dvbuf, dinvfull,
                 idx0, idx1, sidx0, sidx1, rm0, rm1, sm0, sm1,
                 abuf, ambuf, apbuf, sem0, sem1, ssem0, ssem1,
                 dsem0, dsem1, dsem2, dsem3, deg_sh, sx_sh, ix_sh, acc_sh):
    nch = d // 16
    unroll = 4 if nch == 1 else 2
    c = lax.axis_index("c")
    s = lax.axis_index("s")
    wid = c * NS + s
    iota = lax.iota(jnp.int32, 16)
    z16 = jnp.zeros((16,), jnp.int32)

    def zrow(r, carry):
        for ch in range(nch):
            sm0[r, pl.ds(ch * 16, 16)] = jnp.zeros((16,), jnp.float32)
        return carry

    lax.fori_loop(0, 128, zrow, 0)
    for i in range(ROWS_W // 128):
        pltpu.sync_copy(sm0, acc_sh.at[pl.ds(s * ROWS_W + i * 128, 128), :])
    for i in range(ROWS_W // 16):
        dvbuf[pl.ds(i * 16, 16)] = jnp.zeros((16,), jnp.float32)
    pltpu.sync_copy(dvbuf, deg_sh.at[pl.ds(s * ROWS_W, ROWS_W)])
    plsc.subcore_barrier()

    pltpu.sync_copy(q_ref, qbuf)
    pltpu.sync_copy(ew_ref.at[2 * s], etbuf.at[:, pl.ds(0, HE_W)])
    pltpu.sync_copy(ew_ref.at[2 * s + 1], etbuf.at[:, pl.ds(HE_W, HE_W)])

    dsets = ((idxm, valm, idxp, valp, dsem0),
             (idxm1, valm1, idxp1, valp1, dsem1),
             (idxm2, valm2, idxp2, valp2, dsem2),
             (idxm3, valm3, idxp3, valp3, dsem3))

    def group1(gg, carry):
        for p in range(4):
            g = 4 * gg + p
            base = g * 16
            im, vm, ip, vp_b, dsem = dsets[p]

            @pl.when(gg > 0)
            def _():
                pltpu.make_async_copy(vm, deg_sh.at[im], dsem).wait()
                pltpu.make_async_copy(vp_b, deg_sh.at[ip], dsem).wait()

            idxs = [etbuf[j, pl.ds(base, 16)] for j in range(K)]
            ps = [plsc.load_gather(qbuf, [idxs[j]]) for j in range(K)]
            mx, se = ps[0], idxs[0]
            mn, ie = ps[0], idxs[0]
            sarg = jnp.zeros((16,), jnp.int32)
            iarg = jnp.zeros((16,), jnp.int32)
            for j in range(1, K):
                up = ps[j] > mx
                mx = jnp.where(up, ps[j], mx)
                se = jnp.where(up, idxs[j], se)
                sarg = jnp.where(up, j, sarg)
                dn = ps[j] < mn
                mn = jnp.where(dn, ps[j], mn)
                ie = jnp.where(dn, idxs[j], ie)
                iarg = jnp.where(dn, j, iarg)
            sxbuf[pl.ds(base, 16)] = sarg
            ixbuf[pl.ds(base, 16)] = iarg
            nm = jnp.zeros((16,), jnp.float32)
            for j in range(K):
                m = jnp.where((idxs[j] != se) & (idxs[j] != ie), 1.0, 0.0)
                nm = nm + m
                im[pl.ds(j * 16, 16)] = idxs[j]
                vm[pl.ds(j * 16, 16)] = m * (2.0 * INV_C)
            vp = (1.0 + nm) * INV_C
            ip[pl.ds(0, 16)] = se
            vp_b[pl.ds(0, 16)] = vp
            ip[pl.ds(16, 16)] = ie
            vp_b[pl.ds(16, 16)] = vp
            pltpu.async_copy(vm, deg_sh.at[im], dsem, add=True)
            pltpu.async_copy(vp_b, deg_sh.at[ip], dsem, add=True)
        return carry

    lax.fori_loop(0, NG1 // 4, group1, 0)
    for im, vm, ip, vp_b, dsem in dsets:
        pltpu.make_async_copy(vm, deg_sh.at[im], dsem).wait()
        pltpu.make_async_copy(vp_b, deg_sh.at[ip], dsem).wait()
    pltpu.sync_copy(sxbuf.at[pl.ds(0, HE_S)], sx_sh.at[pl.ds(s * HE_S, HE_S)])
    pltpu.sync_copy(ixbuf.at[pl.ds(0, HE_S)], ix_sh.at[pl.ds(s * HE_S, HE_S)])
    plsc.subcore_barrier()

    pltpu.sync_copy(deg_sh.at[pl.ds(s * ROWS_W, ROWS_W)], dvbuf)

    def dloop(i, carry):
        dvbuf[pl.ds(i * 16, 16)] = _rsqrt16(1.0 + dvbuf[pl.ds(i * 16, 16)])
        return carry

    lax.fori_loop(0, ROWS_W // 16, dloop, 0)
    pltpu.sync_copy(dvbuf, deg_sh.at[pl.ds(s * ROWS_W, ROWS_W)])

    @pl.when(c == 0)
    def _():
        pltpu.sync_copy(dvbuf, dinv_ref.at[pl.ds(s * ROWS_W, ROWS_W)])

    plsc.subcore_barrier()
    pltpu.sync_copy(deg_sh, dinvfull)

    pltpu.sync_copy(ew_ref.at[wid], etbuf3)
    pltpu.sync_copy(sx_sh.at[pl.ds(wid * HE_W, HE_W)], sxbuf.at[pl.ds(0, HE_W)])
    pltpu.sync_copy(ix_sh.at[pl.ds(wid * HE_W, HE_W)], ixbuf.at[pl.ds(0, HE_W)])

    def start_gather(g, idx, rm, sem):
        base = g * 16
        for j in range(K):
            idx[pl.ds(j * 16, 16)] = etbuf3[j, pl.ds(base, 16)]
        pltpu.async_copy(hw_ref.at[idx], rm, sem)

    def compute_group(t, g, idx, rm, sm, sidx, ssem):
        base = g * 16
        sx = sxbuf[pl.ds(base, 16)]
        ix = ixbuf[pl.ds(base, 16)]
        se = plsc.load_gather(etbuf3, [sx, base + iota])
        ie = plsc.load_gather(etbuf3, [ix, base + iota])
        for j in range(K):
            vj = etbuf3[j, pl.ds(base, 16)]
            dj = plsc.load_gather(dinvfull, [vj])
            m = jnp.where((vj != se) & (vj != ie), 1.0, 0.0)
            am = m * dj
            ambuf[j, :] = am
            abuf[j, :] = am * INV_C
        dse = plsc.load_gather(dinvfull, [se])
        die = plsc.load_gather(dinvfull, [ie])
        apbuf[0, :] = dse * INV_C
        apbuf[1, :] = die * INV_C
        apbuf[2, :] = dse
        apbuf[3, :] = die

        @pl.when(t > 0)
        def _():
            pltpu.make_async_copy(sm, acc_sh.at[sidx], ssem).wait()

        def he(hu, inner):
            for u in range(unroll):
                h = hu * unroll + u
                s_h = sxbuf[pl.ds(base + h, 16)][0]
                i_h = ixbuf[pl.ds(base + h, 16)][0]
                rs = s_h * 16 + h
                ri = i_h * 16 + h
                h_vec = z16 + h
                a_se = plsc.load_gather(apbuf, [z16, h_vec])
                a_ie = plsc.load_gather(apbuf, [z16 + 1, h_vec])
                dse_b = plsc.load_gather(apbuf, [z16 + 2, h_vec])
                die_b = plsc.load_gather(apbuf, [z16 + 3, h_vec])
                a_j = [plsc.load_gather(abuf, [z16 + j, h_vec])
                       for j in range(K)]
                am_j = [plsc.load_gather(ambuf, [z16 + j, h_vec])
                        for j in range(K)]
                for ch in range(nch):
                    sl = pl.ds(ch * 16, 16)
                    gse = dse_b * rm[rs, sl]
                    gie = die_b * rm[ri, sl]
                    pair = gse + gie
                    msum = jnp.zeros((16,), jnp.float32)
                    for j in range(K):
                        msum = msum + am_j[j] * rm[j * 16 + h, sl]
                    for j in range(K):
                        sm[j * 16 + h, sl] = a_j[j] * pair
                    sm[rs, sl] = a_se * (gie + msum)
                    prev = sm[ri, sl]
                    sm[ri, sl] = prev + a_ie * (gse + msum)
            return inner

        lax.fori_loop(0, 16 // unroll, he, 0)
        for j in range(K):
            sidx[pl.ds(j * 16, 16)] = idx[pl.ds(j * 16, 16)]
        pltpu.async_copy(sm, acc_sh.at[sidx], ssem, add=True)

    start_gather(0, idx0, rm0, sem0)

    def tbody(t, carry):
        g0 = 2 * t
        start_gather(g0 + 1, idx1, rm1, sem1)
        pltpu.make_async_copy(hw_ref.at[idx0], rm0, sem0).wait()
        compute_group(t, g0, idx0, rm0, sm0, sidx0, ssem0)

        @pl.when(t < NG // 2 - 1)
        def _():
            start_gather(g0 + 2, idx0, rm0, sem0)

        pltpu.make_async_copy(hw_ref.at[idx1], rm1, sem1).wait()
        compute_group(t, g0 + 1, idx1, rm1, sm1, sidx1, ssem1)
        return carry

    lax.fori_loop(0, NG // 2, tbody, 0)
    pltpu.make_async_copy(sm0, acc_sh.at[sidx0], ssem0).wait()
    pltpu.make_async_copy(sm1, acc_sh.at[sidx1], ssem1).wait()
    plsc.subcore_barrier()
    pltpu.sync_copy(acc_sh.at[pl.ds(s * ROWS_W, ROWS_W), :],
                    ap_ref.at[c].at[pl.ds(s * ROWS_W, ROWS_W), :])


def _slayer(ew, q, hw, d):
    f = pl.kernel(
        functools.partial(_slayer_body, d),
        out_type=[jax.ShapeDtypeStruct((NC, N_PAD, d), jnp.float32),
                  jax.ShapeDtypeStruct((N_PAD,), jnp.float32)],
        mesh=_MESH,
        compiler_params=pltpu.CompilerParams(needs_layout_passes=False,
                                             use_tc_tiling_on_sc=False),
        scratch_types=[
            pltpu.VMEM((N_PAD,), jnp.float32),
            pltpu.VMEM((K, HE_S), jnp.int32),
            pltpu.VMEM((K, HE_W), jnp.int32),
            pltpu.VMEM((HE_S + 16,), jnp.int32),
            pltpu.VMEM((HE_S + 16,), jnp.int32),
            pltpu.VMEM((K * 16,), jnp.int32),
            pltpu.VMEM((K * 16,), jnp.float32),
            pltpu.VMEM((32,), jnp.int32),
            pltpu.VMEM((32,), jnp.float32),
            pltpu.VMEM((K * 16,), jnp.int32),
            pltpu.VMEM((K * 16,), jnp.float32),
            pltpu.VMEM((32,), jnp.int32),
            pltpu.VMEM((32,), jnp.float32),
            pltpu.VMEM((K * 16,), jnp.int32),
            pltpu.VMEM((K * 16,), jnp.float32),
            pltpu.VMEM((32,), jnp.int32),
            pltpu.VMEM((32,), jnp.float32),
            pltpu.VMEM((K * 16,), jnp.int32),
            pltpu.VMEM((K * 16,), jnp.float32),
            pltpu.VMEM((32,), jnp.int32),
            pltpu.VMEM((32,), jnp.float32),
            pltpu.VMEM((ROWS_W,), jnp.float32),
            pltpu.VMEM((N_PAD,), jnp.float32),
            pltpu.VMEM((K * 16,), jnp.int32),
            pltpu.VMEM((K * 16,), jnp.int32),
            pltpu.VMEM((K * 16,), jnp.int32),
            pltpu.VMEM((K * 16,), jnp.int32),
            pltpu.VMEM((K * 16, d), jnp.float32),
            pltpu.VMEM((K * 16, d), jnp.float32),
            pltpu.VMEM((K * 16, d), jnp.float32),
            pltpu.VMEM((K * 16, d), jnp.float32),
            pltpu.VMEM((K, 16), jnp.float32),
            pltpu.VMEM((K, 16), jnp.float32),
            pltpu.VMEM((4, 16), jnp.float32),
            pltpu.SemaphoreType.DMA,
            pltpu.SemaphoreType.DMA,
            pltpu.SemaphoreType.DMA,
            pltpu.SemaphoreType.DMA,
            pltpu.SemaphoreType.DMA,
            pltpu.SemaphoreType.DMA,
            pltpu.SemaphoreType.DMA,
            pltpu.SemaphoreType.DMA,
            pltpu.VMEM_SHARED((N_PAD,), jnp.float32),
            pltpu.VMEM_SHARED((HE_PAD,), jnp.int32),
            pltpu.VMEM_SHARED((HE_PAD,), jnp.int32),
            pltpu.VMEM_SHARED((N_PAD, d), jnp.float32),
        ],
    )
    return f(ew, q, hw)


def kernel(E, H, W1, b1, W2, b2):
    key = jax.random.key(42)
    rv1 = jax.random.uniform(jax.random.fold_in(key, 0), (D_HID,),
                             dtype=jnp.float32)
    rv2 = jax.random.uniform(jax.random.fold_in(key, 1), (N_CLS,),
                             dtype=jnp.float32)
    h_pad = jnp.zeros((N_PAD, D_IN), jnp.float32).at[:N_NODES].set(H)
    dummy_cols = (jnp.arange(HE_PAD, dtype=jnp.int32) % 16) + N_NODES
    et = jnp.broadcast_to(dummy_cols, (K, HE_PAD))
    et = et.at[:, :N_HE].set(E.T.astype(jnp.int32))
    ew = et.reshape(K, NW, HE_W).transpose(1, 0, 2)

    hw1, q1 = _mmq(h_pad, W1, rv1)
    a1p, dinv1 = _slayer(ew, q1.reshape(N_PAD), hw1, D_HID)
    hw2, q2 = _mid(a1p, hw1, dinv1, b1, W2, rv2)
    a2p, dinv2 = _slayer(ew, q2.reshape(N_PAD), hw2, N_CLS)
    out = _fin(a2p, hw2, dinv2, b2)
    return out[:N_NODES]
